# Initial kernel scaffold; baseline (speedup 1.0000x reference)
import jax
import jax.numpy as jnp
from jax.experimental import pallas as pl


def kernel(features, edge_index, W1, b1, W2, b2):
    raise NotImplementedError("write your pallas kernel here")


# speedup vs baseline: 5.6831x; 5.6831x over previous
import functools

import jax
import jax.numpy as jnp
from jax import lax
from jax.experimental import pallas as pl
from jax.experimental.pallas import tpu as pltpu
from jax.experimental.pallas import tpu_sc as plsc

N_NODES = 10000
N_EDGES = 320000
IN_F = 128
H = 64
NC = 2
NS = 16
NW = NC * NS
EPW = N_EDGES // NW
BLK = 128
NFULL = EPW // BLK
REM = EPW - NFULL * BLK
NPAD = 10240
RPT = NPAD // NS
ZR = 128
DEG_W = 16

_mesh = plsc.VectorSubcoreMesh(core_axis_name="c", subcore_axis_name="s")


@functools.partial(
    pl.kernel,
    mesh=_mesh,
    out_type=jax.ShapeDtypeStruct((NC, NPAD, DEG_W), jnp.float32),
    scratch_types=[
        pltpu.VMEM((BLK,), jnp.int32),
        pltpu.VMEM((REM,), jnp.int32),
        pltpu.VMEM((BLK, DEG_W), jnp.float32),
        pltpu.VMEM((RPT, DEG_W), jnp.float32),
        pltpu.VMEM_SHARED((NPAD, DEG_W), jnp.float32),
        pltpu.SemaphoreType.DMA,
    ],
    compiler_params=pltpu.CompilerParams(use_tc_tiling_on_sc=False),
)
def _deg_kernel(dst_hbm, out_hbm, dst_v, dst_t, ones_v, zbuf, deg_s, sem):
    del sem
    cid = lax.axis_index("c")
    sid = lax.axis_index("s")
    wid = cid * NS + sid

    lane = lax.iota(jnp.int32, 16)
    one_row = jnp.where(lane == 0, 1.0, 0.0).astype(jnp.float32)
    zero_row = jnp.zeros((16,), jnp.float32)

    def fill(i, _):
        ones_v[i, :] = one_row
        return 0

    lax.fori_loop(0, BLK, fill, 0)

    def zfill(i, _):
        zbuf[i, :] = zero_row
        return 0

    lax.fori_loop(0, RPT, zfill, 0)
    pltpu.sync_copy(zbuf, deg_s.at[pl.ds(sid * RPT, RPT)])
    plsc.subcore_barrier()

    base = wid * EPW

    def body(j, _):
        off = pl.multiple_of(base + j * BLK, 8)
        pltpu.sync_copy(dst_hbm.at[pl.ds(off, BLK)], dst_v)
        pltpu.sync_copy(ones_v, deg_s.at[dst_v], add=True)
        return 0

    lax.fori_loop(0, NFULL, body, 0)
    off = pl.multiple_of(base + NFULL * BLK, 8)
    pltpu.sync_copy(dst_hbm.at[pl.ds(off, REM)], dst_t)
    pltpu.sync_copy(ones_v.at[pl.ds(0, REM)], deg_s.at[dst_t], add=True)
    plsc.subcore_barrier()
    pltpu.sync_copy(deg_s.at[pl.ds(sid * RPT, RPT)],
                    out_hbm.at[cid, pl.ds(sid * RPT, RPT)])


@functools.partial(
    pl.kernel,
    mesh=_mesh,
    out_type=jax.ShapeDtypeStruct((NC, NPAD, H), jnp.float32),
    scratch_types=[
        pltpu.VMEM((BLK,), jnp.int32),
        pltpu.VMEM((BLK,), jnp.int32),
        pltpu.VMEM((REM,), jnp.int32),
        pltpu.VMEM((REM,), jnp.int32),
        pltpu.VMEM((BLK, H), jnp.float32),
        pltpu.VMEM((REM, H), jnp.float32),
        pltpu.VMEM((ZR, H), jnp.float32),
        pltpu.VMEM_SHARED((NPAD, H), jnp.float32),
        pltpu.SemaphoreType.DMA,
    ],
    compiler_params=pltpu.CompilerParams(use_tc_tiling_on_sc=False),
)
def _prop_kernel(hh_hbm, src_hbm, dst_hbm, out_hbm,
                 src_v, dst_v, src_t, dst_t, rows, rows_t, zbuf, agg_s, sem):
    cid = lax.axis_index("c")
    sid = lax.axis_index("s")
    wid = cid * NS + sid

    zero_row = jnp.zeros((16,), jnp.float32)

    def zfill(i, _):
        r = i // (H // 16)
        c = i % (H // 16)
        zbuf[r, pl.ds(c * 16, 16)] = zero_row
        return 0

    lax.fori_loop(0, ZR * (H // 16), zfill, 0)

    def zcp(j, _):
        pltpu.sync_copy(zbuf, agg_s.at[pl.ds(sid * RPT + j * ZR, ZR)])
        return 0

    lax.fori_loop(0, RPT // ZR, zcp, 0)
    plsc.subcore_barrier()

    base = wid * EPW

    def body(j, _):
        off = pl.multiple_of(base + j * BLK, 8)
        pltpu.sync_copy(src_hbm.at[pl.ds(off, BLK)], src_v)
        pltpu.async_copy(hh_hbm.at[src_v], rows, sem).wait()
        pltpu.sync_copy(dst_hbm.at[pl.ds(off, BLK)], dst_v)
        pltpu.sync_copy(rows, agg_s.at[dst_v], add=True)
        return 0

    lax.fori_loop(0, NFULL, body, 0)
    off = pl.multiple_of(base + NFULL * BLK, 8)
    pltpu.sync_copy(src_hbm.at[pl.ds(off, REM)], src_t)
    pltpu.async_copy(hh_hbm.at[src_t], rows_t, sem).wait()
    pltpu.sync_copy(dst_hbm.at[pl.ds(off, REM)], dst_t)
    pltpu.sync_copy(rows_t, agg_s.at[dst_t], add=True)
    plsc.subcore_barrier()
    pltpu.sync_copy(agg_s.at[pl.ds(sid * RPT, RPT)],
                    out_hbm.at[cid, pl.ds(sid * RPT, RPT)])


BM = 400


def _tc1_body(x_ref, w_ref, b_ref, d_ref, f0_ref, hh_ref, dinv_ref):
    f0 = jnp.dot(x_ref[...], w_ref[...], preferred_element_type=jnp.float32)
    f0 = jnp.maximum(f0 + b_ref[...], 0.0)
    deg = d_ref[0, :, :1] + d_ref[1, :, :1]
    dinv = lax.rsqrt(jnp.maximum(deg, 1.0))
    f0_ref[...] = f0
    hh_ref[...] = f0 * dinv
    dinv_ref[...] = dinv


def _tc2_body(f0_ref, a_ref, dinv_ref, p1_ref, hh1_ref):
    dinv = dinv_ref[...]
    p1 = f0_ref[...] - (a_ref[0] + a_ref[1]) * dinv
    p1_ref[...] = p1
    hh1_ref[...] = p1 * dinv


def _tc3_body(f0_ref, p1_ref, a_ref, dinv_ref, aw_ref, b2_ref, out_ref):
    p2 = p1_ref[...] - (a_ref[0] + a_ref[1]) * dinv_ref[...]
    acc = jnp.dot(f0_ref[...], aw_ref[0:H, :],
                  preferred_element_type=jnp.float32)
    acc = acc + jnp.dot(p1_ref[...], aw_ref[H:2 * H, :],
                        preferred_element_type=jnp.float32)
    acc = acc + jnp.dot(p2, aw_ref[2 * H:3 * H, :],
                        preferred_element_type=jnp.float32)
    out_ref[...] = acc + b2_ref[...]


def kernel(features, edge_index, W1, b1, W2, b2):
    src = edge_index[0]
    dst = edge_index[1]

    w0, w1_, w2_ = W2[:, :H], W2[:, H:2 * H], W2[:, 2 * H:]
    a0 = 3.0 * w0
    a1 = -3.0 * w0 + 3.0 * w1_
    a2 = 0.75 * w0 - 1.5 * w1_ + 0.75 * w2_
    aw = jnp.concatenate([a0.T, a1.T, a2.T], axis=0)

    degp = _deg_kernel(dst)

    grid = N_NODES // BM
    f0, hh0, dinv = pl.pallas_call(
        _tc1_body,
        grid=(grid,),
        in_specs=[
            pl.BlockSpec((BM, IN_F), lambda i: (i, 0)),
            pl.BlockSpec((IN_F, H), lambda i: (0, 0)),
            pl.BlockSpec((1, H), lambda i: (0, 0)),
            pl.BlockSpec((NC, BM, DEG_W), lambda i: (0, i, 0)),
        ],
        out_specs=[
            pl.BlockSpec((BM, H), lambda i: (i, 0)),
            pl.BlockSpec((BM, H), lambda i: (i, 0)),
            pl.BlockSpec((BM, 1), lambda i: (i, 0)),
        ],
        out_shape=[
            jax.ShapeDtypeStruct((N_NODES, H), jnp.float32),
            jax.ShapeDtypeStruct((N_NODES, H), jnp.float32),
            jax.ShapeDtypeStruct((N_NODES, 1), jnp.float32),
        ],
    )(features, W1.T, b1.reshape(1, H), degp)

    aggp1 = _prop_kernel(hh0, src, dst)

    p1, hh1 = pl.pallas_call(
        _tc2_body,
        grid=(grid,),
        in_specs=[
            pl.BlockSpec((BM, H), lambda i: (i, 0)),
            pl.BlockSpec((NC, BM, H), lambda i: (0, i, 0)),
            pl.BlockSpec((BM, 1), lambda i: (i, 0)),
        ],
        out_specs=[
            pl.BlockSpec((BM, H), lambda i: (i, 0)),
            pl.BlockSpec((BM, H), lambda i: (i, 0)),
        ],
        out_shape=[
            jax.ShapeDtypeStruct((N_NODES, H), jnp.float32),
            jax.ShapeDtypeStruct((N_NODES, H), jnp.float32),
        ],
    )(f0, aggp1, dinv)

    aggp2 = _prop_kernel(hh1, src, dst)

    out = pl.pallas_call(
        _tc3_body,
        grid=(grid,),
        in_specs=[
            pl.BlockSpec((BM, H), lambda i: (i, 0)),
            pl.BlockSpec((BM, H), lambda i: (i, 0)),
            pl.BlockSpec((NC, BM, H), lambda i: (0, i, 0)),
            pl.BlockSpec((BM, 1), lambda i: (i, 0)),
            pl.BlockSpec((3 * H, H), lambda i: (0, 0)),
            pl.BlockSpec((1, H), lambda i: (0, 0)),
        ],
        out_specs=pl.BlockSpec((BM, H), lambda i: (i, 0)),
        out_shape=jax.ShapeDtypeStruct((N_NODES, H), jnp.float32),
    )(f0, p1, aggp2, dinv, aw, b2.reshape(1, H))

    return out
